# Initial kernel scaffold; baseline (speedup 1.0000x reference)
import jax
import jax.numpy as jnp
from jax.experimental import pallas as pl


def kernel(Wx, edge_index):
    raise NotImplementedError("write your pallas kernel here")


# speedup vs baseline: 3.7286x; 3.7286x over previous
import functools

import jax
import jax.numpy as jnp
from jax import lax
from jax.experimental import pallas as pl
from jax.experimental.pallas import tpu as pltpu, tpu_sc as plsc

NUM_NODES = 10000
D_FEAT = 128
NUM_EDGES = 320000

_NC = 2
_NS = 16
_NW = _NC * _NS
_B_PER_W = NUM_EDGES // _NW
_C = 80
_N_CHUNKS = _B_PER_W // _C


def _make_gather2():
    mesh = plsc.VectorSubcoreMesh(core_axis_name="c", subcore_axis_name="s")

    @functools.partial(
        pl.kernel,
        mesh=mesh,
        out_type=[
            jax.ShapeDtypeStruct((NUM_EDGES, D_FEAT), jnp.float32),
            jax.ShapeDtypeStruct((NUM_EDGES, D_FEAT), jnp.float32),
        ],
        scratch_types=[
            pltpu.VMEM((_C,), jnp.int32),
            pltpu.VMEM((_C,), jnp.int32),
            pltpu.VMEM((_C, D_FEAT), jnp.float32),
            pltpu.VMEM((_C, D_FEAT), jnp.float32),
            pltpu.SemaphoreType.DMA,
            pltpu.SemaphoreType.DMA,
        ],
    )
    def gather2(src_hbm, dst_hbm, table_hbm, out_i, out_j,
                idx_s, idx_d, rows_s, rows_d, sem_s, sem_d):
        wid = lax.axis_index("s") * _NC + lax.axis_index("c")
        base = wid * _B_PER_W

        def body(k, carry):
            off = base + k * _C
            pltpu.sync_copy(src_hbm.at[pl.ds(off, _C)], idx_s)
            pltpu.sync_copy(dst_hbm.at[pl.ds(off, _C)], idx_d)
            cp_s = pltpu.async_copy(table_hbm.at[idx_s], rows_s, sem_s)
            cp_d = pltpu.async_copy(table_hbm.at[idx_d], rows_d, sem_d)
            cp_s.wait()
            pltpu.sync_copy(rows_s, out_j.at[pl.ds(off, _C)])
            cp_d.wait()
            pltpu.sync_copy(rows_d, out_i.at[pl.ds(off, _C)])
            return carry

        lax.fori_loop(0, _N_CHUNKS, body, 0)

    return gather2


_gather2 = _make_gather2()


def kernel(Wx, edge_index):
    idx = edge_index.astype(jnp.int32)
    out_i, out_j = _gather2(idx[0], idx[1], Wx)
    return (out_i, out_j)


# speedup vs baseline: 6.1050x; 1.6373x over previous
import functools

import jax
import jax.numpy as jnp
from jax import lax
from jax.experimental import pallas as pl
from jax.experimental.pallas import tpu as pltpu, tpu_sc as plsc

NUM_NODES = 10000
D_FEAT = 128
NUM_EDGES = 320000

_NC = 2
_NS = 16
_NW = _NC * _NS
_B_PER_W = NUM_EDGES // _NW
_C = 80
_N_CHUNKS = _B_PER_W // _C
_NBUF = 5
_G = _N_CHUNKS // _NBUF


def _make_gather2():
    mesh = plsc.VectorSubcoreMesh(core_axis_name="c", subcore_axis_name="s")

    scratch = [
        pltpu.VMEM((_B_PER_W,), jnp.int32),
        pltpu.VMEM((_B_PER_W,), jnp.int32),
        pltpu.VMEM((_NBUF, _C, D_FEAT), jnp.float32),
        pltpu.VMEM((_NBUF, _C, D_FEAT), jnp.float32),
    ] + [pltpu.SemaphoreType.DMA] * (4 * _NBUF)

    @functools.partial(
        pl.kernel,
        mesh=mesh,
        out_type=[
            jax.ShapeDtypeStruct((NUM_EDGES, D_FEAT), jnp.float32),
            jax.ShapeDtypeStruct((NUM_EDGES, D_FEAT), jnp.float32),
        ],
        scratch_types=scratch,
    )
    def gather2(src_hbm, dst_hbm, table_hbm, out_i, out_j, *scr):
        idx_s, idx_d, rows_s, rows_d = scr[0:4]
        sem_gs = scr[4:4 + _NBUF]
        sem_gd = scr[4 + _NBUF:4 + 2 * _NBUF]
        sem_ws = scr[4 + 2 * _NBUF:4 + 3 * _NBUF]
        sem_wd = scr[4 + 3 * _NBUF:4 + 4 * _NBUF]

        wid = lax.axis_index("s") * _NC + lax.axis_index("c")
        base = wid * _B_PER_W

        pltpu.sync_copy(src_hbm.at[wid], idx_s)
        pltpu.sync_copy(dst_hbm.at[wid], idx_d)

        def start_gather(ch, b):
            sl = pl.ds(ch * _C, _C)
            pltpu.async_copy(table_hbm.at[idx_s.at[sl]], rows_s.at[b], sem_gs[b])
            pltpu.async_copy(table_hbm.at[idx_d.at[sl]], rows_d.at[b], sem_gd[b])

        def wait_gather(b):
            sl = pl.ds(0, _C)
            pltpu.make_async_copy(table_hbm.at[idx_s.at[sl]], rows_s.at[b], sem_gs[b]).wait()
            pltpu.make_async_copy(table_hbm.at[idx_d.at[sl]], rows_d.at[b], sem_gd[b]).wait()

        def start_writeout(ch, b):
            off = base + ch * _C
            pltpu.async_copy(rows_s.at[b], out_j.at[pl.ds(off, _C)], sem_ws[b])
            pltpu.async_copy(rows_d.at[b], out_i.at[pl.ds(off, _C)], sem_wd[b])

        def wait_writeout(b):
            pltpu.make_async_copy(rows_s.at[b], out_j.at[pl.ds(0, _C)], sem_ws[b]).wait()
            pltpu.make_async_copy(rows_d.at[b], out_i.at[pl.ds(0, _C)], sem_wd[b]).wait()

        for b in range(_NBUF):
            start_gather(b, b)

        for b in range(_NBUF):
            if b > 0:
                wait_writeout(b - 1)
                start_gather(b - 1 + _NBUF, b - 1)
            wait_gather(b)
            start_writeout(b, b)

        def body(g, carry):
            ch0 = g * _NBUF
            for b in range(_NBUF):
                ch = ch0 + b
                pb = (b + _NBUF - 1) % _NBUF
                wait_writeout(pb)
                start_gather(ch - 1 + _NBUF, pb)
                wait_gather(b)
                start_writeout(ch, b)
            return carry

        lax.fori_loop(1, _G - 1, body, 0)

        ch0 = (_G - 1) * _NBUF
        for b in range(_NBUF):
            ch = ch0 + b
            pb = (b + _NBUF - 1) % _NBUF
            wait_writeout(pb)
            if b == 0:
                start_gather(ch - 1 + _NBUF, pb)
            wait_gather(b)
            start_writeout(ch, b)

        wait_writeout((_N_CHUNKS - 1) % _NBUF)

    return gather2


_gather2 = _make_gather2()


def kernel(Wx, edge_index):
    idx = edge_index.astype(jnp.int32).reshape(2, _NW, _B_PER_W)
    out_i, out_j = _gather2(idx[0], idx[1], Wx)
    return (out_i, out_j)


# speedup vs baseline: 6.1198x; 1.0024x over previous
import functools

import jax
import jax.numpy as jnp
from jax import lax
from jax.experimental import pallas as pl
from jax.experimental.pallas import tpu as pltpu, tpu_sc as plsc

NUM_NODES = 10000
D_FEAT = 128
NUM_EDGES = 320000

_NC = 2
_NS = 16
_NW = _NC * _NS
_B_PER_W = NUM_EDGES // _NW
_C = 80
_N_CHUNKS = _B_PER_W // _C
_NBUF = 5
_G = _N_CHUNKS // _NBUF
_LAG = 2


def _make_gather2():
    mesh = plsc.VectorSubcoreMesh(core_axis_name="c", subcore_axis_name="s")

    scratch = [
        pltpu.VMEM((_B_PER_W,), jnp.int32),
        pltpu.VMEM((_B_PER_W,), jnp.int32),
        pltpu.VMEM((_NBUF, _C, D_FEAT), jnp.float32),
        pltpu.VMEM((_NBUF, _C, D_FEAT), jnp.float32),
    ] + [pltpu.SemaphoreType.DMA] * (4 * _NBUF)

    @functools.partial(
        pl.kernel,
        mesh=mesh,
        out_type=[
            jax.ShapeDtypeStruct((NUM_EDGES, D_FEAT), jnp.float32),
            jax.ShapeDtypeStruct((NUM_EDGES, D_FEAT), jnp.float32),
        ],
        scratch_types=scratch,
    )
    def gather2(src_hbm, dst_hbm, table_hbm, out_i, out_j, *scr):
        idx_s, idx_d, rows_s, rows_d = scr[0:4]
        sem_gs = scr[4:4 + _NBUF]
        sem_gd = scr[4 + _NBUF:4 + 2 * _NBUF]
        sem_ws = scr[4 + 2 * _NBUF:4 + 3 * _NBUF]
        sem_wd = scr[4 + 3 * _NBUF:4 + 4 * _NBUF]

        wid = lax.axis_index("s") * _NC + lax.axis_index("c")
        base = wid * _B_PER_W

        pltpu.sync_copy(src_hbm.at[wid], idx_s)
        pltpu.sync_copy(dst_hbm.at[wid], idx_d)

        def start_gather(ch, b):
            sl = pl.ds(ch * _C, _C)
            pltpu.async_copy(table_hbm.at[idx_s.at[sl]], rows_s.at[b], sem_gs[b])
            pltpu.async_copy(table_hbm.at[idx_d.at[sl]], rows_d.at[b], sem_gd[b])

        def wait_gather(b):
            sl = pl.ds(0, _C)
            pltpu.make_async_copy(table_hbm.at[idx_s.at[sl]], rows_s.at[b], sem_gs[b]).wait()
            pltpu.make_async_copy(table_hbm.at[idx_d.at[sl]], rows_d.at[b], sem_gd[b]).wait()

        def start_writeout(ch, b):
            off = base + ch * _C
            pltpu.async_copy(rows_s.at[b], out_j.at[pl.ds(off, _C)], sem_ws[b])
            pltpu.async_copy(rows_d.at[b], out_i.at[pl.ds(off, _C)], sem_wd[b])

        def wait_writeout(b):
            pltpu.make_async_copy(rows_s.at[b], out_j.at[pl.ds(0, _C)], sem_ws[b]).wait()
            pltpu.make_async_copy(rows_d.at[b], out_i.at[pl.ds(0, _C)], sem_wd[b]).wait()

        for b in range(_NBUF):
            start_gather(b, b)

        for b in range(_NBUF):
            if b >= _LAG:
                wait_writeout((b - _LAG) % _NBUF)
                start_gather(b - _LAG + _NBUF, (b - _LAG) % _NBUF)
            wait_gather(b)
            start_writeout(b, b)

        def body(g, carry):
            ch0 = g * _NBUF
            for b in range(_NBUF):
                ch = ch0 + b
                pb = (b + _NBUF - _LAG) % _NBUF
                wait_writeout(pb)
                start_gather(ch - _LAG + _NBUF, pb)
                wait_gather(b)
                start_writeout(ch, b)
            return carry

        lax.fori_loop(1, _G - 1, body, 0)

        ch0 = (_G - 1) * _NBUF
        for b in range(_NBUF):
            ch = ch0 + b
            pb = (b + _NBUF - _LAG) % _NBUF
            wait_writeout(pb)
            if ch - _LAG + _NBUF <= _N_CHUNKS - 1:
                start_gather(ch - _LAG + _NBUF, pb)
            wait_gather(b)
            start_writeout(ch, b)

        for ch in range(_N_CHUNKS - _LAG, _N_CHUNKS):
            wait_writeout(ch % _NBUF)

    return gather2


_gather2 = _make_gather2()


def kernel(Wx, edge_index):
    idx = edge_index.astype(jnp.int32).reshape(2, _NW, _B_PER_W)
    out_i, out_j = _gather2(idx[0], idx[1], Wx)
    return (out_i, out_j)
